# Initial kernel scaffold; baseline (speedup 1.0000x reference)
import jax
import jax.numpy as jnp
from jax.experimental import pallas as pl


def kernel(x, edge_index, W, b):
    raise NotImplementedError("write your pallas kernel here")


# speedup vs baseline: 19.8132x; 19.8132x over previous
import functools

import jax
import jax.numpy as jnp
from jax import lax
from jax.experimental import pallas as pl
from jax.experimental.pallas import tpu as pltpu
from jax.experimental.pallas import tpu_sc as plsc

N = 10000
E = 320000
D = 128

NC = 2
NS = 16
NW = NC * NS
CH = 128
C = -(-E // (NW * CH))
E_PAD = NW * C * CH
N_PAD = 10240
ZB = N_PAD // NS
ZR = 32
TRASH = N

_mesh = plsc.VectorSubcoreMesh(
    core_axis_name="c", subcore_axis_name="s", num_cores=NC, num_subcores=NS)


@functools.partial(
    pl.kernel, mesh=_mesh,
    out_type=jax.ShapeDtypeStruct((NC, N_PAD), jnp.float32),
    scratch_types=[
        pltpu.VMEM((C, CH), jnp.int32),
        pltpu.VMEM((CH,), jnp.float32),
        pltpu.VMEM((ZB,), jnp.float32),
        pltpu.VMEM_SHARED((N_PAD,), jnp.float32),
    ],
)
def _deg_kernel(col_hbm, deg_out, col_v, ones_v, zeros_v, deg_sp):
    cid = lax.axis_index("c")
    sid = lax.axis_index("s")
    wid = cid * NS + sid

    one = jnp.ones((16,), jnp.float32)
    zero = jnp.zeros((16,), jnp.float32)

    def fill_ones(i, _):
        ones_v[pl.ds(i * 16, 16)] = one
        return 0
    lax.fori_loop(0, CH // 16, fill_ones, 0)

    def fill_zeros(i, _):
        zeros_v[pl.ds(i * 16, 16)] = zero
        return 0
    lax.fori_loop(0, ZB // 16, fill_zeros, 0)

    pltpu.sync_copy(zeros_v, deg_sp.at[pl.ds(sid * ZB, ZB)])
    plsc.subcore_barrier()

    pltpu.sync_copy(col_hbm.at[wid], col_v)

    def body(j, _):
        pltpu.sync_copy(ones_v, deg_sp.at[col_v.at[j]], add=True)
        return 0
    lax.fori_loop(0, C, body, 0)

    plsc.subcore_barrier()
    pltpu.sync_copy(deg_sp.at[pl.ds(sid * ZB, ZB)],
                    deg_out.at[cid, pl.ds(sid * ZB, ZB)])


@functools.partial(
    pl.kernel, mesh=_mesh,
    out_type=jax.ShapeDtypeStruct((NC, N_PAD, D), jnp.float32),
    scratch_types=[
        pltpu.VMEM((C, CH), jnp.int32),
        pltpu.VMEM((C, CH), jnp.int32),
        pltpu.VMEM((CH, D), jnp.float32),
        pltpu.VMEM((ZR, D), jnp.float32),
        pltpu.VMEM_SHARED((N_PAD, D), jnp.float32),
        pltpu.SemaphoreType.DMA,
    ],
)
def _agg_kernel(y_hbm, row_hbm, col_hbm, s_out, row_v, col_v, buf, zbuf, s_sp, sem):
    cid = lax.axis_index("c")
    sid = lax.axis_index("s")
    wid = cid * NS + sid

    zero = jnp.zeros((16,), jnp.float32)

    def fill_zeros(r, _):
        for q in range(D // 16):
            zbuf[r, pl.ds(q * 16, 16)] = zero
        return 0
    lax.fori_loop(0, ZR, fill_zeros, 0)

    for t in range(ZB // ZR):
        pltpu.sync_copy(zbuf, s_sp.at[pl.ds(sid * ZB + t * ZR, ZR)])
    plsc.subcore_barrier()

    pltpu.sync_copy(row_hbm.at[wid], row_v)
    pltpu.sync_copy(col_hbm.at[wid], col_v)

    def body(j, _):
        pltpu.async_copy(y_hbm.at[row_v.at[j]], buf, sem).wait()
        pltpu.sync_copy(buf, s_sp.at[col_v.at[j]], add=True)
        return 0
    lax.fori_loop(0, C, body, 0)

    plsc.subcore_barrier()
    for t in range(ZB // ZR):
        off = sid * ZB + t * ZR
        pltpu.sync_copy(s_sp.at[pl.ds(off, ZR)], s_out.at[cid, pl.ds(off, ZR)])


def _transform_body(x_ref, w_ref, dp_ref, y_ref, dis_ref):
    deg = dp_ref[0, :] + dp_ref[1, :] + 1.0
    dis = lax.rsqrt(deg)
    dis_ref[0, :] = dis
    xw = jnp.dot(x_ref[...], w_ref[...], preferred_element_type=jnp.float32)
    y_ref[...] = xw * dis[:, None]


def _finalize_body(sp_ref, y_ref, dis_ref, b_ref, o_ref):
    s = sp_ref[0] + sp_ref[1] + y_ref[...]
    o_ref[...] = jnp.maximum(s * dis_ref[0, :][:, None] + b_ref[0, :], 0.0)


def kernel(x, edge_index, W, b):
    row = edge_index[0].astype(jnp.int32)
    col = edge_index[1].astype(jnp.int32)
    pad = E_PAD - E
    row_c = jnp.concatenate([row, jnp.zeros((pad,), jnp.int32)]).reshape(NW, C, CH)
    col_c = jnp.concatenate([col, jnp.full((pad,), TRASH, jnp.int32)]).reshape(NW, C, CH)
    x_pad = jnp.pad(x, ((0, N_PAD - N), (0, 0)))

    deg_p = _deg_kernel(col_c)

    blk = 1024
    y, dis = pl.pallas_call(
        _transform_body,
        grid=(N_PAD // blk,),
        in_specs=[
            pl.BlockSpec((blk, D), lambda i: (i, 0)),
            pl.BlockSpec((D, D), lambda i: (0, 0)),
            pl.BlockSpec((NC, blk), lambda i: (0, i)),
        ],
        out_specs=[
            pl.BlockSpec((blk, D), lambda i: (i, 0)),
            pl.BlockSpec((1, blk), lambda i: (0, i)),
        ],
        out_shape=[
            jax.ShapeDtypeStruct((N_PAD, D), jnp.float32),
            jax.ShapeDtypeStruct((1, N_PAD), jnp.float32),
        ],
    )(x_pad, W, deg_p)

    s_p = _agg_kernel(y, row_c, col_c)

    out = pl.pallas_call(
        _finalize_body,
        grid=(N_PAD // blk,),
        in_specs=[
            pl.BlockSpec((NC, blk, D), lambda i: (0, i, 0)),
            pl.BlockSpec((blk, D), lambda i: (i, 0)),
            pl.BlockSpec((1, blk), lambda i: (0, i)),
            pl.BlockSpec((1, D), lambda i: (0, 0)),
        ],
        out_specs=pl.BlockSpec((blk, D), lambda i: (i, 0)),
        out_shape=jax.ShapeDtypeStruct((N_PAD, D), jnp.float32),
    )(s_p, y, dis, b.reshape(1, D))
    return out[:N]
